# baseline (speedup 1.0000x reference)
import functools

import jax
import jax.numpy as jnp
from jax import lax
from jax.experimental import pallas as pl
from jax.experimental.pallas import tpu as pltpu


def _sorter_body(w_ref, b_ref, x_ref, val_ref, id_ref):
    w = w_ref[...]
    xb = x_ref[...]
    val = jnp.dot(w, xb, preferred_element_type=jnp.float32)
    val = val + b_ref[...]
    val_ref[...] = val
    R, N = val.shape
    mx = jnp.max(val, axis=0, keepdims=True)
    iota = lax.broadcasted_iota(jnp.int32, (R, N), 0)
    ids = jnp.min(jnp.where(val == mx, iota, jnp.int32(2**30)), axis=0,
                  keepdims=True)
    id_ref[...] = ids


def _sorter(x, w2d, b2d):
    B, F, N = x.shape
    R = w2d.shape[0]
    val, ids = pl.pallas_call(
        _sorter_body,
        grid=(B,),
        in_specs=[
            pl.BlockSpec((R, F), lambda b: (0, 0)),
            pl.BlockSpec((R, 1), lambda b: (0, 0)),
            pl.BlockSpec((None, F, N), lambda b: (b, 0, 0)),
        ],
        out_specs=[
            pl.BlockSpec((None, R, N), lambda b: (b, 0, 0)),
            pl.BlockSpec((None, 1, N), lambda b: (b, 0, 0)),
        ],
        out_shape=[
            jax.ShapeDtypeStruct((B, R, N), jnp.float32),
            jax.ShapeDtypeStruct((B, 1, N), jnp.int32),
        ],
    )(w2d, b2d, x)
    return val, ids.reshape(B, N)


def kernel(x, w_sorter, b_sorter, w1, b1, w2, b2, w3, b3, w5, b5):
    B, F, N = x.shape
    R = w_sorter.shape[0]
    sp_ratio = 4
    num_cabin = 8
    pnt = N // sp_ratio

    val_activa, id_activa = _sorter(x, w_sorter[:, :, 0],
                                    b_sorter.reshape(R, 1))

    order = jnp.argsort(-val_activa, axis=2)
    idx = order[:, :, :pnt]
    idx_b = jnp.broadcast_to(idx[:, None, :, :], (B, F, R, pnt))
    sp_cube = jnp.take_along_axis(x[:, :, None, :], idx_b, axis=3)
    sp_idx = jnp.broadcast_to(idx[:, None, :, :].astype(jnp.float32),
                              (B, R + 3, R, pnt))
    points_cabin = pnt // num_cabin
    cabins = jnp.max(sp_cube.reshape(B, F, R, num_cabin, points_cabin), axis=4)
    return (sp_cube, sp_idx, cabins, id_activa)


# speedup vs baseline: 792.3532x; 792.3532x over previous
import functools

import jax
import jax.numpy as jnp
from jax import lax
from jax.experimental import pallas as pl
from jax.experimental.pallas import tpu as pltpu
from jax.experimental.pallas import tpu_sc as plsc

B, F, N = 16, 256, 2048
R, PNT = 16, 512
NF_HALF = F // 2


def _sorter_body(w_ref, b_ref, x_ref, val_ref, id_ref):
    w = w_ref[...]
    xb = x_ref[...]
    val = jnp.dot(w, xb, preferred_element_type=jnp.float32)
    val = val + b_ref[...]
    val_ref[...] = val
    mx = jnp.max(val, axis=0, keepdims=True)
    iota = lax.broadcasted_iota(jnp.int32, (R, N), 0)
    ids = jnp.min(jnp.where(val == mx, iota, jnp.int32(2**30)), axis=0,
                  keepdims=True)
    id_ref[...] = ids


def _sorter(x, w2d, b2d):
    val, ids = pl.pallas_call(
        _sorter_body,
        grid=(B,),
        in_specs=[
            pl.BlockSpec((R, F), lambda b: (0, 0)),
            pl.BlockSpec((R, 1), lambda b: (0, 0)),
            pl.BlockSpec((None, F, N), lambda b: (b, 0, 0)),
        ],
        out_specs=[
            pl.BlockSpec((None, R, N), lambda b: (b, 0, 0)),
            pl.BlockSpec((None, 1, N), lambda b: (b, 0, 0)),
        ],
        out_shape=[
            jax.ShapeDtypeStruct((B, R, N), jnp.float32),
            jax.ShapeDtypeStruct((B, 1, N), jnp.int32),
        ],
    )(w2d, b2d, x)
    return val, ids.reshape(B, N)


def _gather_body(x_hbm, idx_hbm, cube_hbm, cab_hbm,
                 idx_v, x_row, out_row, cab_part, cab_out):
    c = lax.axis_index("c")
    s = lax.axis_index("s")
    w = s * 2 + c
    b = w // 2
    fhalf = w % 2

    pltpu.sync_copy(idx_hbm.at[b], idx_v)

    def f_loop(fi, _):
        row = b * F + fhalf * NF_HALF + fi
        pltpu.sync_copy(x_hbm.at[row], x_row)

        def rc_loop(it, _):
            r = it // 8
            c4 = it % 8
            base = r * PNT + c4 * 64
            cmax = jnp.full((16,), -jnp.inf, jnp.float32)
            for q in range(4):
                off = base + q * 16
                iv = idx_v[pl.ds(off, 16)]
                g = plsc.load_gather(x_row, [iv])
                out_row[pl.ds(off, 16)] = g
                cmax = jnp.maximum(cmax, g)
            cab_part[pl.ds(it * 16, 16)] = cmax
            return 0

        lax.fori_loop(0, R * 8, rc_loop, 0)

        def tr_loop(j, _):
            acc = jnp.full((16,), -jnp.inf, jnp.float32)
            col = lax.iota(jnp.int32, 16) * 16 + j * 16 * 16
            for l in range(16):
                v = plsc.load_gather(cab_part, [col + l])
                acc = jnp.maximum(acc, v)
            cab_out[pl.ds(j * 16, 16)] = acc
            return 0

        lax.fori_loop(0, 8, tr_loop, 0)

        pltpu.sync_copy(out_row, cube_hbm.at[row])
        pltpu.sync_copy(cab_out, cab_hbm.at[row])
        return 0

    lax.fori_loop(0, NF_HALF, f_loop, 0)


def _sc_gather(x2d, idxflat):
    mesh = plsc.VectorSubcoreMesh(core_axis_name="c", subcore_axis_name="s")
    return pl.kernel(
        _gather_body,
        out_type=[
            jax.ShapeDtypeStruct((B * F, R * PNT), jnp.float32),
            jax.ShapeDtypeStruct((B * F, R * 8), jnp.float32),
        ],
        mesh=mesh,
        scratch_types=[
            pltpu.VMEM((R * PNT,), jnp.int32),
            pltpu.VMEM((N,), jnp.float32),
            pltpu.VMEM((R * PNT,), jnp.float32),
            pltpu.VMEM((R * 8 * 16,), jnp.float32),
            pltpu.VMEM((R * 8,), jnp.float32),
        ],
        compiler_params=pltpu.CompilerParams(needs_layout_passes=False),
    )(x2d, idxflat)


def kernel(x, w_sorter, b_sorter, w1, b1, w2, b2, w3, b3, w5, b5):
    val_activa, id_activa = _sorter(x, w_sorter[:, :, 0],
                                    b_sorter.reshape(R, 1))

    order = jnp.argsort(-val_activa, axis=2)
    idx = order[:, :, :PNT].astype(jnp.int32)

    cube, cab = _sc_gather(x.reshape(B * F, N), idx.reshape(B, R * PNT))
    sp_cube = cube.reshape(B, F, R, PNT)
    cabins = cab.reshape(B, F, R, 8)
    sp_idx = jnp.broadcast_to(idx[:, None, :, :].astype(jnp.float32),
                              (B, R + 3, R, PNT))
    return (sp_cube, sp_idx, cabins, id_activa)
